# baseline (speedup 1.0000x reference)
import functools

import jax
import jax.numpy as jnp
from jax import lax
from jax.experimental import pallas as pl
from jax.experimental.pallas import tpu as pltpu
from jax.experimental.pallas import tpu_sc as plsc

N = 10000
E = 320000
NTILES = 16
NCORES = 2
CH = 128
SPC = 80
NSTEPS = 160
NB = 5
PF = 3
NSA = NSTEPS + 4
NACC = 10240
RPT = NACC // NTILES


def _make_scat(H):
    mesh = plsc.VectorSubcoreMesh(core_axis_name="c", subcore_axis_name="s")

    @functools.partial(
        pl.kernel,
        mesh=mesh,
        out_type=jax.ShapeDtypeStruct((NCORES, NACC, H), jnp.float32),
        compiler_params=pltpu.CompilerParams(use_tc_tiling_on_sc=False),
        scratch_types=[
            pltpu.VMEM((CH,), jnp.int32),
            pltpu.VMEM((CH,), jnp.int32),
            pltpu.VMEM((CH, H), jnp.float32),
            pltpu.VMEM_SHARED((NACC, H), jnp.float32),
            pltpu.SemaphoreType.DMA,
        ],
    )
    def kern(src, rowi, coli, out, rbuf, cbuf, dbuf, acc, gsem):
        c = lax.axis_index("c")
        s = lax.axis_index("s")

        zero = jnp.zeros((16,), jnp.float32)

        def zrow(i, carry):
            for j in range(H // 16):
                dbuf[i, pl.ds(j * 16, 16)] = zero
            return carry

        lax.fori_loop(0, CH, zrow, 0)
        for k in range(RPT // CH):
            pltpu.sync_copy(dbuf, acc.at[pl.ds(s * RPT + k * CH, CH)])
        plsc.subcore_barrier()

        def step(i, carry):
            pltpu.sync_copy(coli.at[s, i], cbuf)
            pltpu.sync_copy(rowi.at[c, s, i], rbuf)
            pltpu.async_copy(src.at[rbuf], dbuf, gsem).wait()
            pltpu.sync_copy(dbuf, acc.at[cbuf], add=True)
            return carry

        lax.fori_loop(0, NSTEPS, step, 0)
        plsc.subcore_barrier()

        pltpu.sync_copy(acc.at[pl.ds(s * RPT, RPT)],
                        out.at[c, pl.ds(s * RPT, RPT)])

    return kern


def _make_degree():
    H = 16
    mesh = plsc.VectorSubcoreMesh(core_axis_name="c", subcore_axis_name="s")

    @functools.partial(
        pl.kernel,
        mesh=mesh,
        out_type=jax.ShapeDtypeStruct((NCORES, NACC, H), jnp.float32),
        compiler_params=pltpu.CompilerParams(use_tc_tiling_on_sc=False),
        scratch_types=[
            pltpu.VMEM((SPC, CH), jnp.int32),
            pltpu.VMEM((CH, H), jnp.float32),
            pltpu.VMEM_SHARED((NACC, H), jnp.float32),
            pltpu.SemaphoreType.DMA,
        ],
    )
    def kern(coli, out, cbuf, dbuf, acc, sem):
        c = lax.axis_index("c")
        s = lax.axis_index("s")

        pltpu.sync_copy(coli.at[s, pl.ds(c * SPC, SPC)], cbuf)

        zero = jnp.zeros((16,), jnp.float32)

        def zrow(i, carry):
            dbuf[i, pl.ds(0, 16)] = zero
            return carry

        lax.fori_loop(0, CH, zrow, 0)
        for k in range(RPT // CH):
            pltpu.sync_copy(dbuf, acc.at[pl.ds(s * RPT + k * CH, CH)])
        plsc.subcore_barrier()

        one = jnp.ones((16,), jnp.float32)

        def orow(i, carry):
            dbuf[i, pl.ds(0, 16)] = one
            return carry

        lax.fori_loop(0, CH, orow, 0)

        W = 8

        def scat(i):
            return pltpu.make_async_copy(dbuf, acc.at[cbuf.at[i]], sem)

        for i in range(W):
            scat(i).start(add=True)

        def step(i, carry):
            scat(i).wait()
            scat(i + W).start(add=True)
            return carry

        lax.fori_loop(0, SPC - W, step, 0)
        for i in range(SPC - W, SPC):
            scat(i).wait()
        plsc.subcore_barrier()

        pltpu.sync_copy(acc.at[pl.ds(s * RPT, RPT)],
                        out.at[c, pl.ds(s * RPT, RPT)])

    return kern


_BLK = 400


def _split_scale(y, dv, alpha, Wy):
    K = y.shape[1]

    def body(y_r, dv_r, o_r):
        sc = alpha * dv_r[...]
        yv = y_r[...]
        for h in range(2):
            o_r[h] = sc * yv[:, h * Wy:(h + 1) * Wy]

    return pl.pallas_call(
        body,
        grid=(N // _BLK,),
        in_specs=[
            pl.BlockSpec((_BLK, K), lambda i: (i, 0)),
            pl.BlockSpec((_BLK, 1), lambda i: (i, 0)),
        ],
        out_specs=pl.BlockSpec((2, _BLK, Wy), lambda i: (0, i, 0)),
        out_shape=jax.ShapeDtypeStruct((2, N, Wy), jnp.float32),
    )(y, dv)


def _halves_scale(y, dv, alpha, Wy):
    def body(y_r, dv_r, o_r):
        sc = alpha * dv_r[...]
        for h in range(2):
            o_r[h] = sc * y_r[h]

    return pl.pallas_call(
        body,
        grid=(N // _BLK,),
        in_specs=[
            pl.BlockSpec((2, _BLK, Wy), lambda i: (0, i, 0)),
            pl.BlockSpec((_BLK, 1), lambda i: (i, 0)),
        ],
        out_specs=pl.BlockSpec((2, _BLK, Wy), lambda i: (0, i, 0)),
        out_shape=jax.ShapeDtypeStruct((2, N, Wy), jnp.float32),
    )(y, dv)


def _cheb_layer(z, y1, y2, dv, W0, W1, W2, b, Bn, want_xu):
    K = z.shape[1]
    Wy = K // 2
    F = W0.shape[1]
    Hn = F // 2

    def body(z_r, y1_r, y2_r, dv_r, w0_r, w1_r, w2_r, b_r, bn_r, x_r,
             xu_r=None):
        dv = dv_r[...]
        zv = z_r[...]
        o = jnp.dot(zv, w0_r[...], preferred_element_type=jnp.float32)
        w1 = w1_r[...]
        w2 = w2_r[...]
        for h in range(2):
            sl = slice(h * Wy, (h + 1) * Wy)
            tx1h = -dv * y1_r[h]
            o += jnp.dot(tx1h, w1[sl], preferred_element_type=jnp.float32)
            tx2h = -2.0 * dv * y2_r[h] - zv[:, sl]
            o += jnp.dot(tx2h, w2[sl], preferred_element_type=jnp.float32)
        o += b_r[...]
        x = jnp.maximum(o, 0.0) * bn_r[...]
        x_r[...] = x
        if want_xu:
            for h in range(2):
                xu_r[h] = x[:, h * Hn:(h + 1) * Hn] * dv

    yspec = pl.BlockSpec((2, _BLK, Wy), lambda i: (0, i, 0))
    in_specs = [
        pl.BlockSpec((_BLK, K), lambda i: (i, 0)),
        yspec,
        yspec,
        pl.BlockSpec((_BLK, 1), lambda i: (i, 0)),
        pl.BlockSpec((K, F), lambda i: (0, 0)),
        pl.BlockSpec((K, F), lambda i: (0, 0)),
        pl.BlockSpec((K, F), lambda i: (0, 0)),
        pl.BlockSpec((1, F), lambda i: (0, 0)),
        pl.BlockSpec((_BLK, F), lambda i: (i, 0)),
    ]
    out_shape = [jax.ShapeDtypeStruct((N, F), jnp.float32)]
    out_specs = [pl.BlockSpec((_BLK, F), lambda i: (i, 0))]
    if want_xu:
        out_shape.append(jax.ShapeDtypeStruct((2, N, Hn), jnp.float32))
        out_specs.append(pl.BlockSpec((2, _BLK, Hn), lambda i: (0, i, 0)))

    res = pl.pallas_call(
        body,
        grid=(N // _BLK,),
        in_specs=in_specs,
        out_specs=out_specs,
        out_shape=out_shape,
    )(z, y1, y2, dv, W0, W1, W2, b, Bn)
    if want_xu:
        return res
    return res[0], None


def _pad_w(W, K):
    k = W.shape[1]
    if k == K:
        return W
    return jnp.pad(W, ((0, 0), (0, K - k), (0, 0)))


def kernel(v, edges, W1, b1, W2, b2, W3, b3):
    row = edges[0].reshape(NTILES, NCORES, E // NTILES // NCORES)
    col = edges[1].reshape(NTILES, NCORES, E // NTILES // NCORES)
    padn = SPC * CH - E // NTILES // NCORES
    r3 = jnp.concatenate(
        [row, jnp.zeros((NTILES, NCORES, padn), jnp.int32)],
        axis=2).reshape(NTILES, NSTEPS, CH)
    r3 = jnp.concatenate(
        [r3, jnp.zeros((NTILES, NSA - NSTEPS, CH), jnp.int32)], axis=1)
    coli = jnp.concatenate(
        [col, jnp.full((NTILES, NCORES, padn), N, jnp.int32)],
        axis=2).reshape(NTILES, NSTEPS, CH)
    rowt = jnp.concatenate(
        [row, jnp.full((NTILES, NCORES, padn), N, jnp.int32)],
        axis=2).reshape(NTILES, NSTEPS, CH)
    rowi = jnp.stack([r3, r3 + N])

    dacc = _make_degree()(rowt)
    deg = dacc[0, :N, 0] + dacc[1, :N, 0]
    dv = jnp.where(deg > 0, lax.rsqrt(deg), 0.0).reshape(N, 1)

    nkey = jax.random.key(42)
    B1n = jax.random.normal(jax.random.fold_in(nkey, 1), (N, 128), jnp.float32)
    B2n = jax.random.normal(jax.random.fold_in(nkey, 2), (N, 256), jnp.float32)
    B3n = jax.random.normal(jax.random.fold_in(nkey, 3), (N, 512), jnp.float32)

    scat64 = _make_scat(64)
    scat128 = _make_scat(128)

    def prop_pair(u, Wy, scat):
        y1 = scat(u.reshape(NCORES * N, Wy), rowi, coli)
        u2 = _halves_scale(y1, dv * dv, -1.0, Wy)
        y2 = scat(u2.reshape(NCORES * N, Wy), rowi, coli)
        return y1, y2

    zp = jnp.pad(v, ((0, 0), (0, 128 - 86)))
    u = _split_scale(zp, dv, 1.0, 64)
    y1, y2 = prop_pair(u, 64, scat64)
    Wp = _pad_w(W1, 128)
    x1, xu = _cheb_layer(zp, y1, y2, dv, Wp[0], Wp[1], Wp[2],
                         b1.reshape(1, -1), B1n, True)

    y1, y2 = prop_pair(xu, 64, scat64)
    x2, xu = _cheb_layer(x1, y1, y2, dv, W2[0], W2[1], W2[2],
                         b2.reshape(1, -1), B2n, True)

    y1, y2 = prop_pair(xu, 128, scat128)
    x3, _ = _cheb_layer(x2, y1, y2, dv, W3[0], W3[1], W3[2],
                        b3.reshape(1, -1), B3n, False)

    return (x1, x2, x3)


# speedup vs baseline: 1.2720x; 1.2720x over previous
import functools

import jax
import jax.numpy as jnp
from jax import lax
from jax.experimental import pallas as pl
from jax.experimental.pallas import tpu as pltpu
from jax.experimental.pallas import tpu_sc as plsc

N = 10000
E = 320000
NTILES = 16
NCORES = 2
CH = 128
EPT = E // NTILES
NSTEPS = 157
PT = NSTEPS * CH
EPAD = NTILES * PT
NACC = 10240
RPT = NACC // NTILES


def _make_scatter(H):
    mesh = plsc.VectorSubcoreMesh(core_axis_name="c", subcore_axis_name="s")

    @functools.partial(
        pl.kernel,
        mesh=mesh,
        out_type=jax.ShapeDtypeStruct((NCORES, NACC, H), jnp.float32),
        compiler_params=pltpu.CompilerParams(use_tc_tiling_on_sc=False),
        scratch_types=[
            pltpu.VMEM((CH,), jnp.int32),
            pltpu.VMEM((CH,), jnp.int32),
            pltpu.VMEM((CH, H), jnp.float32),
            pltpu.VMEM_SHARED((NACC, H), jnp.float32),
            pltpu.SemaphoreType.DMA,
        ],
    )
    def kern(src, rowi, coli, out, rbuf, cbuf, dbuf, acc, gsem):
        c = lax.axis_index("c")
        s = lax.axis_index("s")

        zero = jnp.zeros((16,), jnp.float32)

        def zrow(i, carry):
            for j in range(H // 16):
                dbuf[i, pl.ds(j * 16, 16)] = zero
            return carry

        lax.fori_loop(0, CH, zrow, 0)
        for k in range(RPT // CH):
            pltpu.sync_copy(dbuf, acc.at[pl.ds(s * RPT + k * CH, CH)])
        plsc.subcore_barrier()

        base = s * PT

        def step(i, carry):
            off = base + i * CH
            pltpu.sync_copy(coli.at[pl.ds(off, CH)], cbuf)
            pltpu.sync_copy(rowi.at[c, pl.ds(off, CH)], rbuf)
            pltpu.async_copy(src.at[rbuf], dbuf, gsem).wait()
            pltpu.sync_copy(dbuf, acc.at[cbuf], add=True)
            return carry

        lax.fori_loop(0, NSTEPS, step, 0)
        plsc.subcore_barrier()

        pltpu.sync_copy(acc.at[pl.ds(s * RPT, RPT)],
                        out.at[c, pl.ds(s * RPT, RPT)])

    return kern


def _make_degree():
    H = 16
    mesh = plsc.VectorSubcoreMesh(core_axis_name="c", subcore_axis_name="s")

    @functools.partial(
        pl.kernel,
        mesh=mesh,
        out_type=jax.ShapeDtypeStruct((NCORES, NACC, H), jnp.float32),
        compiler_params=pltpu.CompilerParams(use_tc_tiling_on_sc=False),
        scratch_types=[
            pltpu.VMEM((CH,), jnp.int32),
            pltpu.VMEM((CH, H), jnp.float32),
            pltpu.VMEM_SHARED((NACC, H), jnp.float32),
        ],
    )
    def kern(coli, out, cbuf, dbuf, acc):
        c = lax.axis_index("c")
        s = lax.axis_index("s")

        zero = jnp.zeros((16,), jnp.float32)

        def zrow(i, carry):
            dbuf[i, pl.ds(0, 16)] = zero
            return carry

        lax.fori_loop(0, CH, zrow, 0)
        for k in range(RPT // CH):
            pltpu.sync_copy(dbuf, acc.at[pl.ds(s * RPT + k * CH, CH)])
        plsc.subcore_barrier()

        one = jnp.ones((16,), jnp.float32)

        def orow(i, carry):
            dbuf[i, pl.ds(0, 16)] = one
            return carry

        lax.fori_loop(0, CH, orow, 0)

        base = s * PT

        def step(i, carry):
            off = base + i * CH
            pltpu.sync_copy(coli.at[pl.ds(off, CH)], cbuf)
            pltpu.sync_copy(dbuf, acc.at[cbuf], add=True)
            return carry

        lax.fori_loop(0, NSTEPS, step, 0)
        plsc.subcore_barrier()

        pltpu.sync_copy(acc.at[pl.ds(s * RPT, RPT)],
                        out.at[c, pl.ds(s * RPT, RPT)])

    return kern


_BLK = 400


def _split_scale(y, dv, alpha, H):
    K = y.shape[1]

    def body(y_r, dv_r, o_r):
        s = alpha * dv_r[...]
        yv = y_r[...]
        for h in range(2):
            o_r[h] = s * yv[:, h * H:(h + 1) * H]

    return pl.pallas_call(
        body,
        grid=(N // _BLK,),
        in_specs=[
            pl.BlockSpec((_BLK, K), lambda i: (i, 0)),
            pl.BlockSpec((_BLK, 1), lambda i: (i, 0)),
        ],
        out_specs=pl.BlockSpec((2, _BLK, H), lambda i: (0, i, 0)),
        out_shape=jax.ShapeDtypeStruct((2, N, H), jnp.float32),
    )(y, dv)


def _halves_scale(y, dv, alpha, H):
    def body(y_r, dv_r, o_r):
        s = alpha * dv_r[...]
        for h in range(2):
            o_r[h] = s * y_r[h]

    return pl.pallas_call(
        body,
        grid=(N // _BLK,),
        in_specs=[
            pl.BlockSpec((2, _BLK, H), lambda i: (0, i, 0)),
            pl.BlockSpec((_BLK, 1), lambda i: (i, 0)),
        ],
        out_specs=pl.BlockSpec((2, _BLK, H), lambda i: (0, i, 0)),
        out_shape=jax.ShapeDtypeStruct((2, N, H), jnp.float32),
    )(y, dv)


def _cheb_layer(z, y1, y2, dv, W0, W1, W2, b, Bn, want_xu):
    K = z.shape[1]
    H = K // 2
    F = W0.shape[1]
    Hn = F // 2

    def body(z_r, y1_r, y2_r, dv_r, w0_r, w1_r, w2_r, b_r, bn_r, x_r,
             xu_r=None):
        dv = dv_r[...]
        zv = z_r[...]
        o = jnp.dot(zv, w0_r[...], preferred_element_type=jnp.float32)
        w1 = w1_r[...]
        w2 = w2_r[...]
        for h in range(2):
            sl = slice(h * H, (h + 1) * H)
            tx1h = -dv * y1_r[h]
            o += jnp.dot(tx1h, w1[sl], preferred_element_type=jnp.float32)
            tx2h = -2.0 * dv * y2_r[h] - zv[:, sl]
            o += jnp.dot(tx2h, w2[sl], preferred_element_type=jnp.float32)
        o += b_r[...]
        x = jnp.maximum(o, 0.0) * bn_r[...]
        x_r[...] = x
        if want_xu:
            for h in range(2):
                xu_r[h] = x[:, h * Hn:(h + 1) * Hn] * dv

    in_specs = [
        pl.BlockSpec((_BLK, K), lambda i: (i, 0)),
        pl.BlockSpec((2, _BLK, H), lambda i: (0, i, 0)),
        pl.BlockSpec((2, _BLK, H), lambda i: (0, i, 0)),
        pl.BlockSpec((_BLK, 1), lambda i: (i, 0)),
        pl.BlockSpec((K, F), lambda i: (0, 0)),
        pl.BlockSpec((K, F), lambda i: (0, 0)),
        pl.BlockSpec((K, F), lambda i: (0, 0)),
        pl.BlockSpec((1, F), lambda i: (0, 0)),
        pl.BlockSpec((_BLK, F), lambda i: (i, 0)),
    ]
    out_shape = [jax.ShapeDtypeStruct((N, F), jnp.float32)]
    out_specs = [pl.BlockSpec((_BLK, F), lambda i: (i, 0))]
    if want_xu:
        out_shape.append(jax.ShapeDtypeStruct((2, N, Hn), jnp.float32))
        out_specs.append(pl.BlockSpec((2, _BLK, Hn), lambda i: (0, i, 0)))

    res = pl.pallas_call(
        body,
        grid=(N // _BLK,),
        in_specs=in_specs,
        out_specs=out_specs,
        out_shape=out_shape,
    )(z, y1, y2, dv, W0, W1, W2, b, Bn)
    if want_xu:
        return res
    return res[0], None


def _pad_w(W, K):
    k = W.shape[1]
    if k == K:
        return W
    return jnp.pad(W, ((0, 0), (0, K - k), (0, 0)))


def kernel(v, edges, W1, b1, W2, b2, W3, b3):
    row = edges[0].reshape(NTILES, EPT)
    col = edges[1].reshape(NTILES, EPT)
    padz = jnp.zeros((NTILES, PT - EPT), jnp.int32)
    padt = jnp.full((NTILES, PT - EPT), N, jnp.int32)
    rowp = jnp.concatenate([row, padz], axis=1).reshape(-1)
    rowt = jnp.concatenate([row, padt], axis=1).reshape(-1)
    colt = jnp.concatenate([col, padt], axis=1).reshape(-1)
    rowi = jnp.stack([rowp, rowp + N])
    coli = colt

    deg = _make_degree()(rowt)[0, :N, 0]
    dv = jnp.where(deg > 0, lax.rsqrt(deg), 0.0).reshape(N, 1)

    nkey = jax.random.key(42)
    B1n = jax.random.normal(jax.random.fold_in(nkey, 1), (N, 128), jnp.float32)
    B2n = jax.random.normal(jax.random.fold_in(nkey, 2), (N, 256), jnp.float32)
    B3n = jax.random.normal(jax.random.fold_in(nkey, 3), (N, 512), jnp.float32)

    scat64 = _make_scatter(64)
    scat128 = _make_scatter(128)

    def prop_pair(u_split, H, scat):
        src1 = u_split.reshape(NCORES * N, H)
        y1 = scat(src1, rowi, coli)
        u2 = _halves_scale(y1, dv * dv, -1.0, H)
        y2 = scat(u2.reshape(NCORES * N, H), rowi, coli)
        return y1, y2

    zp = jnp.pad(v, ((0, 0), (0, 128 - 86)))
    u = _split_scale(zp, dv, 1.0, 64)
    y1, y2 = prop_pair(u, 64, scat64)
    Wp = _pad_w(W1, 128)
    x1, xu = _cheb_layer(zp, y1, y2, dv, Wp[0], Wp[1], Wp[2],
                         b1.reshape(1, -1), B1n, True)

    y1, y2 = prop_pair(xu, 64, scat64)
    x2, xu = _cheb_layer(x1, y1, y2, dv, W2[0], W2[1], W2[2],
                         b2.reshape(1, -1), B2n, True)

    y1, y2 = prop_pair(xu, 128, scat128)
    x3, _ = _cheb_layer(x2, y1, y2, dv, W3[0], W3[1], W3[2],
                        b3.reshape(1, -1), B3n, False)

    return (x1, x2, x3)


# speedup vs baseline: 1.4756x; 1.1600x over previous
import functools

import jax
import jax.numpy as jnp
from jax import lax
from jax.experimental import pallas as pl
from jax.experimental.pallas import tpu as pltpu
from jax.experimental.pallas import tpu_sc as plsc

N = 10000
E = 320000
NTILES = 16
NCORES = 2
CH = 128
EPT = E // NTILES
NSTEPS = 157
PT = NSTEPS * CH
EPAD = NTILES * PT
NACC = 10240
RPT = NACC // NTILES


def _make_scatter(H):
    mesh = plsc.VectorSubcoreMesh(core_axis_name="c", subcore_axis_name="s")

    @functools.partial(
        pl.kernel,
        mesh=mesh,
        out_type=jax.ShapeDtypeStruct((NCORES, NACC, H), jnp.float32),
        compiler_params=pltpu.CompilerParams(use_tc_tiling_on_sc=False),
        scratch_types=[
            pltpu.VMEM((CH,), jnp.int32),
            pltpu.VMEM((CH,), jnp.int32),
            pltpu.VMEM((CH,), jnp.int32),
            pltpu.VMEM((CH, H), jnp.float32),
            pltpu.VMEM_SHARED((NACC, H), jnp.float32),
            pltpu.SemaphoreType.DMA,
        ],
    )
    def kern(src, pki, out, pbuf, rbuf, cbuf, dbuf, acc, gsem):
        c = lax.axis_index("c")
        s = lax.axis_index("s")
        roff = c * N

        zero = jnp.zeros((16,), jnp.float32)

        def zrow(i, carry):
            for j in range(H // 16):
                dbuf[i, pl.ds(j * 16, 16)] = zero
            return carry

        lax.fori_loop(0, CH, zrow, 0)
        for k in range(RPT // CH):
            pltpu.sync_copy(dbuf, acc.at[pl.ds(s * RPT + k * CH, CH)])
        plsc.subcore_barrier()

        base = s * PT

        def step(i, carry):
            off = base + i * CH
            pltpu.sync_copy(pki.at[pl.ds(off, CH)], pbuf)
            for k in range(CH // 16):
                d = pl.ds(k * 16, 16)
                p = pbuf[d]
                rbuf[d] = (p & 0x7FFF) + roff
                cbuf[d] = p >> 15
            pltpu.async_copy(src.at[rbuf], dbuf, gsem).wait()
            pltpu.sync_copy(dbuf, acc.at[cbuf], add=True)
            return carry

        lax.fori_loop(0, NSTEPS, step, 0)
        plsc.subcore_barrier()

        pltpu.sync_copy(acc.at[pl.ds(s * RPT, RPT)],
                        out.at[c, pl.ds(s * RPT, RPT)])

    return kern


def _make_degree():
    H = 16
    mesh = plsc.VectorSubcoreMesh(core_axis_name="c", subcore_axis_name="s")

    @functools.partial(
        pl.kernel,
        mesh=mesh,
        out_type=jax.ShapeDtypeStruct((NCORES, NACC, H), jnp.float32),
        compiler_params=pltpu.CompilerParams(use_tc_tiling_on_sc=False),
        scratch_types=[
            pltpu.VMEM((CH,), jnp.int32),
            pltpu.VMEM((CH, H), jnp.float32),
            pltpu.VMEM_SHARED((NACC, H), jnp.float32),
        ],
    )
    def kern(coli, out, cbuf, dbuf, acc):
        c = lax.axis_index("c")
        s = lax.axis_index("s")

        zero = jnp.zeros((16,), jnp.float32)

        def zrow(i, carry):
            dbuf[i, pl.ds(0, 16)] = zero
            return carry

        lax.fori_loop(0, CH, zrow, 0)
        for k in range(RPT // CH):
            pltpu.sync_copy(dbuf, acc.at[pl.ds(s * RPT + k * CH, CH)])
        plsc.subcore_barrier()

        one = jnp.ones((16,), jnp.float32)

        def orow(i, carry):
            dbuf[i, pl.ds(0, 16)] = one
            return carry

        lax.fori_loop(0, CH, orow, 0)

        base = s * PT

        def step(i, carry):
            off = base + i * CH
            pltpu.sync_copy(coli.at[pl.ds(off, CH)], cbuf)
            pltpu.sync_copy(dbuf, acc.at[cbuf], add=True)
            return carry

        lax.fori_loop(0, NSTEPS, step, 0)
        plsc.subcore_barrier()

        pltpu.sync_copy(acc.at[pl.ds(s * RPT, RPT)],
                        out.at[c, pl.ds(s * RPT, RPT)])

    return kern


_BLK = 400


def _split_scale(y, dv, alpha, H):
    K = y.shape[1]

    def body(y_r, dv_r, o_r):
        s = alpha * dv_r[...]
        yv = y_r[...]
        for h in range(2):
            o_r[h] = s * yv[:, h * H:(h + 1) * H]

    return pl.pallas_call(
        body,
        grid=(N // _BLK,),
        in_specs=[
            pl.BlockSpec((_BLK, K), lambda i: (i, 0)),
            pl.BlockSpec((_BLK, 1), lambda i: (i, 0)),
        ],
        out_specs=pl.BlockSpec((2, _BLK, H), lambda i: (0, i, 0)),
        out_shape=jax.ShapeDtypeStruct((2, N, H), jnp.float32),
    )(y, dv)


def _halves_scale(y, dv, alpha, H):
    def body(y_r, dv_r, o_r):
        s = alpha * dv_r[...]
        for h in range(2):
            o_r[h] = s * y_r[h]

    return pl.pallas_call(
        body,
        grid=(N // _BLK,),
        in_specs=[
            pl.BlockSpec((2, _BLK, H), lambda i: (0, i, 0)),
            pl.BlockSpec((_BLK, 1), lambda i: (i, 0)),
        ],
        out_specs=pl.BlockSpec((2, _BLK, H), lambda i: (0, i, 0)),
        out_shape=jax.ShapeDtypeStruct((2, N, H), jnp.float32),
    )(y, dv)


def _cheb_layer(z, y1, y2, dv, W0, W1, W2, b, Bn, want_xu):
    K = z.shape[1]
    H = K // 2
    F = W0.shape[1]
    Hn = F // 2

    def body(z_r, y1_r, y2_r, dv_r, w0_r, w1_r, w2_r, b_r, bn_r, x_r,
             xu_r=None):
        dv = dv_r[...]
        zv = z_r[...]
        o = jnp.dot(zv, w0_r[...], preferred_element_type=jnp.float32)
        w1 = w1_r[...]
        w2 = w2_r[...]
        for h in range(2):
            sl = slice(h * H, (h + 1) * H)
            tx1h = -dv * y1_r[h]
            o += jnp.dot(tx1h, w1[sl], preferred_element_type=jnp.float32)
            tx2h = -2.0 * dv * y2_r[h] - zv[:, sl]
            o += jnp.dot(tx2h, w2[sl], preferred_element_type=jnp.float32)
        o += b_r[...]
        x = jnp.maximum(o, 0.0) * bn_r[...]
        x_r[...] = x
        if want_xu:
            for h in range(2):
                xu_r[h] = x[:, h * Hn:(h + 1) * Hn] * dv

    in_specs = [
        pl.BlockSpec((_BLK, K), lambda i: (i, 0)),
        pl.BlockSpec((2, _BLK, H), lambda i: (0, i, 0)),
        pl.BlockSpec((2, _BLK, H), lambda i: (0, i, 0)),
        pl.BlockSpec((_BLK, 1), lambda i: (i, 0)),
        pl.BlockSpec((K, F), lambda i: (0, 0)),
        pl.BlockSpec((K, F), lambda i: (0, 0)),
        pl.BlockSpec((K, F), lambda i: (0, 0)),
        pl.BlockSpec((1, F), lambda i: (0, 0)),
        pl.BlockSpec((_BLK, F), lambda i: (i, 0)),
    ]
    out_shape = [jax.ShapeDtypeStruct((N, F), jnp.float32)]
    out_specs = [pl.BlockSpec((_BLK, F), lambda i: (i, 0))]
    if want_xu:
        out_shape.append(jax.ShapeDtypeStruct((2, N, Hn), jnp.float32))
        out_specs.append(pl.BlockSpec((2, _BLK, Hn), lambda i: (0, i, 0)))

    res = pl.pallas_call(
        body,
        grid=(N // _BLK,),
        in_specs=in_specs,
        out_specs=out_specs,
        out_shape=out_shape,
    )(z, y1, y2, dv, W0, W1, W2, b, Bn)
    if want_xu:
        return res
    return res[0], None


def _pad_w(W, K):
    k = W.shape[1]
    if k == K:
        return W
    return jnp.pad(W, ((0, 0), (0, K - k), (0, 0)))


def kernel(v, edges, W1, b1, W2, b2, W3, b3):
    row = edges[0].reshape(NTILES, EPT)
    col = edges[1].reshape(NTILES, EPT)
    padz = jnp.zeros((NTILES, PT - EPT), jnp.int32)
    padt = jnp.full((NTILES, PT - EPT), N, jnp.int32)
    rowp = jnp.concatenate([row, padz], axis=1).reshape(-1)
    rowt = jnp.concatenate([row, padt], axis=1).reshape(-1)
    colt = jnp.concatenate([col, padt], axis=1).reshape(-1)
    pki = rowp | (colt << 15)

    deg = _make_degree()(rowt)[0, :N, 0]
    dv = jnp.where(deg > 0, lax.rsqrt(deg), 0.0).reshape(N, 1)

    nkey = jax.random.key(42)
    B1n = jax.random.normal(jax.random.fold_in(nkey, 1), (N, 128), jnp.float32)
    B2n = jax.random.normal(jax.random.fold_in(nkey, 2), (N, 256), jnp.float32)
    B3n = jax.random.normal(jax.random.fold_in(nkey, 3), (N, 512), jnp.float32)

    scat64 = _make_scatter(64)
    scat128 = _make_scatter(128)

    def prop_pair(u_split, H, scat):
        src1 = u_split.reshape(NCORES * N, H)
        y1 = scat(src1, pki)
        u2 = _halves_scale(y1, dv * dv, -1.0, H)
        y2 = scat(u2.reshape(NCORES * N, H), pki)
        return y1, y2

    zp = jnp.pad(v, ((0, 0), (0, 128 - 86)))
    u = _split_scale(zp, dv, 1.0, 64)
    y1, y2 = prop_pair(u, 64, scat64)
    Wp = _pad_w(W1, 128)
    x1, xu = _cheb_layer(zp, y1, y2, dv, Wp[0], Wp[1], Wp[2],
                         b1.reshape(1, -1), B1n, True)

    y1, y2 = prop_pair(xu, 64, scat64)
    x2, xu = _cheb_layer(x1, y1, y2, dv, W2[0], W2[1], W2[2],
                         b2.reshape(1, -1), B2n, True)

    y1, y2 = prop_pair(xu, 128, scat128)
    x3, _ = _cheb_layer(x2, y1, y2, dv, W3[0], W3[1], W3[2],
                        b3.reshape(1, -1), B3n, False)

    return (x1, x2, x3)
